# Initial kernel scaffold; baseline (speedup 1.0000x reference)
import functools

import jax
import jax.numpy as jnp
from jax import lax
from jax.experimental import pallas as pl
from jax.experimental.pallas import tpu as pltpu
from jax.experimental.pallas import tpu_sc as plsc

_LANES = 16
_IDXW = 128


@functools.lru_cache(maxsize=None)
def _build(V1, D, B, NC, NS):
    NW = NC * NS
    CH = 1024
    K = CH // _IDXW
    b_per_w = B // NW
    n_chunks = b_per_w // CH
    assert b_per_w % CH == 0

    mesh = plsc.VectorSubcoreMesh(core_axis_name="c", subcore_axis_name="s")

    @functools.partial(
        pl.kernel,
        mesh=mesh,
        out_type=jax.ShapeDtypeStruct((B, D), jnp.float32),
        scratch_types=[
            pltpu.VMEM((K, _IDXW), jnp.int32),
            pltpu.VMEM((CH, D), jnp.float32),
            pltpu.SemaphoreType.DMA,
        ],
    )
    def emb(idx_hbm, table_hbm, out_hbm, idx_v, rows_v, sem):
        wid = lax.axis_index("s") * NC + lax.axis_index("c")
        irow0 = wid * (b_per_w // _IDXW)
        orow0 = wid * b_per_w

        def chunk(g, carry):
            pltpu.sync_copy(idx_hbm.at[pl.ds(irow0 + g * K, K)], idx_v)
            cps = [
                pltpu.async_copy(
                    table_hbm.at[idx_v.at[j]],
                    rows_v.at[pl.ds(j * _IDXW, _IDXW)],
                    sem,
                )
                for j in range(K)
            ]
            for cp in cps:
                cp.wait()
            for j in range(K):
                for o in range(_IDXW // _LANES):
                    v = idx_v[j, pl.ds(o * _LANES, _LANES)]
                    m = v == 0
                    anyz = jnp.max(m.astype(jnp.int32))

                    @pl.when(anyz > 0)
                    def _zero(j=j, o=o, m=m):
                        rows = lax.iota(jnp.int32, _LANES) + (j * _IDXW + o * _LANES)
                        z = jnp.zeros((_LANES,), jnp.float32)

                        def zcol(c, acc):
                            cols = jnp.zeros((_LANES,), jnp.int32) + c
                            plsc.store_scatter(rows_v, [rows, cols], z, mask=m)
                            return acc

                        lax.fori_loop(0, D, zcol, 0)

            pltpu.sync_copy(rows_v, out_hbm.at[pl.ds(orow0 + g * CH, CH)])
            return carry

        lax.fori_loop(0, n_chunks, chunk, 0)

    return emb


def kernel(feature_ids, table):
    batch, hist = feature_ids.shape
    V1, D = table.shape
    B = batch * hist
    ids = feature_ids.reshape(-1).astype(jnp.int32).reshape(B // _IDXW, _IDXW)
    info = plsc.get_sparse_core_info()
    emb = _build(V1, D, B, info.num_cores, info.num_subcores)
    out = emb(ids, table)
    return out.reshape(batch, hist, D)


# speedup vs baseline: 4.5171x; 4.5171x over previous
import functools

import jax
import jax.numpy as jnp
from jax import lax
from jax.experimental import pallas as pl
from jax.experimental.pallas import tpu as pltpu
from jax.experimental.pallas import tpu_sc as plsc

_LANES = 16
_IDXW = 128


@functools.lru_cache(maxsize=None)
def _build(V1, D, B, NC, NS):
    NW = NC * NS
    CH = 1024
    K = CH // _IDXW
    b_per_w = B // NW
    n_chunks = b_per_w // CH
    assert b_per_w % CH == 0

    mesh = plsc.VectorSubcoreMesh(core_axis_name="c", subcore_axis_name="s")

    @functools.partial(
        pl.kernel,
        mesh=mesh,
        compiler_params=pltpu.CompilerParams(use_tc_tiling_on_sc=False, needs_layout_passes=False),
        out_type=jax.ShapeDtypeStruct((B, D), jnp.float32),
        scratch_types=[
            pltpu.VMEM((K, _IDXW), jnp.int32),
            pltpu.VMEM((CH, D), jnp.float32),
            pltpu.SemaphoreType.DMA,
        ],
    )
    def emb(idx_hbm, table_hbm, out_hbm, idx_v, rows_v, sem):
        wid = lax.axis_index("s") * NC + lax.axis_index("c")
        irow0 = wid * (b_per_w // _IDXW)
        orow0 = wid * b_per_w

        def chunk(g, carry):
            pltpu.sync_copy(idx_hbm.at[pl.ds(irow0 + g * K, K)], idx_v)
            cps = [
                pltpu.async_copy(
                    table_hbm.at[idx_v.at[j]],
                    rows_v.at[pl.ds(j * _IDXW, _IDXW)],
                    sem,
                )
                for j in range(K)
            ]
            for cp in cps:
                cp.wait()
            for j in range(K):
                for o in range(_IDXW // _LANES):
                    v = idx_v[j, pl.ds(o * _LANES, _LANES)]
                    anyz = jnp.min(v)

                    @pl.when(anyz == 0)
                    def _zero(j=j, o=o, v=v):
                        m = v == 0
                        rows = lax.iota(jnp.int32, _LANES) + (j * _IDXW + o * _LANES)
                        z = jnp.zeros((_LANES,), jnp.float32)

                        def zcol(c, acc):
                            cols = jnp.zeros((_LANES,), jnp.int32) + c
                            plsc.store_scatter(rows_v, [rows, cols], z, mask=m)
                            return acc

                        lax.fori_loop(0, D, zcol, 0)

            pltpu.sync_copy(rows_v, out_hbm.at[pl.ds(orow0 + g * CH, CH)])
            return carry

        lax.fori_loop(0, n_chunks, chunk, 0)

    return emb


def kernel(feature_ids, table):
    batch, hist = feature_ids.shape
    V1, D = table.shape
    B = batch * hist
    ids = feature_ids.reshape(-1).astype(jnp.int32).reshape(B // _IDXW, _IDXW)
    info = plsc.get_sparse_core_info()
    emb = _build(V1, D, B, info.num_cores, info.num_subcores)
    out = emb(ids, table)
    return out.reshape(batch, hist, D)


# speedup vs baseline: 5.0587x; 1.1199x over previous
import functools

import jax
import jax.numpy as jnp
from jax import lax
from jax.experimental import pallas as pl
from jax.experimental.pallas import tpu as pltpu
from jax.experimental.pallas import tpu_sc as plsc

_LANES = 16
_IDXW = 128


@functools.lru_cache(maxsize=None)
def _build(V1, D, B, NC, NS):
    NW = NC * NS
    CH = 1280
    K = CH // _IDXW
    b_per_w = B // NW
    n_chunks = b_per_w // CH
    assert b_per_w % CH == 0 and n_chunks % 2 == 0

    mesh = plsc.VectorSubcoreMesh(core_axis_name="c", subcore_axis_name="s")

    @functools.partial(
        pl.kernel,
        mesh=mesh,
        compiler_params=pltpu.CompilerParams(
            use_tc_tiling_on_sc=False, needs_layout_passes=False),
        out_type=jax.ShapeDtypeStruct((B, D), jnp.float32),
        scratch_types=[
            pltpu.VMEM((2, K, _IDXW), jnp.int32),
            pltpu.VMEM((2, CH, D), jnp.float32),
            pltpu.SemaphoreType.DMA,
            pltpu.SemaphoreType.DMA,
            pltpu.SemaphoreType.DMA,
            pltpu.SemaphoreType.DMA,
            pltpu.SemaphoreType.DMA,
        ],
    )
    def emb(idx_hbm, table_hbm, out_hbm, idx_v, rows_v, gat_sem,
            i_sem0, i_sem1, o_sem0, o_sem1):
        wid = lax.axis_index("s") * NC + lax.axis_index("c")
        irow0 = wid * (b_per_w // _IDXW)
        orow0 = wid * b_per_w
        i_sems = (i_sem0, i_sem1)
        o_sems = (o_sem0, o_sem1)

        for b in range(2):
            pltpu.async_copy(
                idx_hbm.at[pl.ds(irow0 + b * K, K)], idx_v.at[b], i_sems[b])

        def outer(t, carry):
            for b in range(2):
                g = 2 * t + b

                @pl.when(t > 0)
                def _wait_store(b=b):
                    pltpu.make_async_copy(
                        rows_v.at[b], out_hbm.at[pl.ds(orow0, CH)],
                        o_sems[b]).wait()

                pltpu.make_async_copy(
                    idx_hbm.at[pl.ds(irow0, K)], idx_v.at[b],
                    i_sems[b]).wait()

                cps = [
                    pltpu.async_copy(
                        table_hbm.at[idx_v.at[b].at[j]],
                        rows_v.at[b].at[pl.ds(j * _IDXW, _IDXW)],
                        gat_sem,
                    )
                    for j in range(K)
                ]

                mn = idx_v[b, 0, pl.ds(0, _LANES)]
                for j in range(K):
                    for o in range(_IDXW // _LANES):
                        if j == 0 and o == 0:
                            continue
                        mn = jnp.minimum(mn, idx_v[b, j, pl.ds(o * _LANES, _LANES)])
                has_pad = jnp.min(mn) == 0

                for cp in cps:
                    cp.wait()

                @pl.when(has_pad)
                def _fixup(b=b):
                    for j in range(K):
                        for o in range(_IDXW // _LANES):
                            v = idx_v[b, j, pl.ds(o * _LANES, _LANES)]

                            @pl.when(jnp.min(v) == 0)
                            def _zero(b=b, j=j, o=o, v=v):
                                m = v == 0
                                rows = lax.iota(jnp.int32, _LANES) + (
                                    j * _IDXW + o * _LANES)
                                z = jnp.zeros((_LANES,), jnp.float32)

                                def zcol(c, acc):
                                    cols = jnp.zeros((_LANES,), jnp.int32) + c
                                    plsc.store_scatter(
                                        rows_v.at[b], [rows, cols], z, mask=m)
                                    return acc

                                lax.fori_loop(0, D, zcol, 0)

                @pl.when(g + 2 < n_chunks)
                def _prefetch(b=b, g=g):
                    pltpu.async_copy(
                        idx_hbm.at[pl.ds(irow0 + (g + 2) * K, K)],
                        idx_v.at[b], i_sems[b])

                pltpu.async_copy(
                    rows_v.at[b], out_hbm.at[pl.ds(orow0 + g * CH, CH)],
                    o_sems[b])
            return carry

        lax.fori_loop(0, n_chunks // 2, outer, 0)

        for b in range(2):
            pltpu.make_async_copy(
                rows_v.at[b], out_hbm.at[pl.ds(orow0, CH)], o_sems[b]).wait()

    return emb


def kernel(feature_ids, table):
    batch, hist = feature_ids.shape
    V1, D = table.shape
    B = batch * hist
    ids = feature_ids.reshape(-1).astype(jnp.int32).reshape(B // _IDXW, _IDXW)
    info = plsc.get_sparse_core_info()
    emb = _build(V1, D, B, info.num_cores, info.num_subcores)
    out = emb(ids, table)
    return out.reshape(batch, hist, D)
